# Initial kernel scaffold; baseline (speedup 1.0000x reference)
import jax
import jax.numpy as jnp
from jax.experimental import pallas as pl


def kernel(output1, output2, quant):
    raise NotImplementedError("write your pallas kernel here")


# speedup vs baseline: 9.2643x; 9.2643x over previous
import functools

import jax
import jax.numpy as jnp
from jax.experimental import pallas as pl

_N = 4096
_D = 512
_BLK = 256


def _body(o1_ref, o2t_ref, rn_ref, chosen_ref, pos_ref):
    r0 = pl.program_id(0) * _BLK
    o1 = o1_ref[...]
    o2t = o2t_ref[...]
    n1 = jnp.sum(o1 * o1, axis=1, keepdims=True)
    n2 = jnp.sum(o2t * o2t, axis=0, keepdims=True)
    dot = jnp.dot(o1, o2t, preferred_element_type=jnp.float32)
    d2 = n1 - 2.0 * dot + n2
    cols = jax.lax.broadcasted_iota(jnp.int32, (_BLK, _N), 1)
    rows = jax.lax.broadcasted_iota(jnp.int32, (_BLK, _N), 0) + r0
    diag = cols == rows
    d2 = jnp.where(diag, jnp.inf, d2)
    d2 = jnp.maximum(d2, 1e-12)
    xbits = jax.lax.bitcast_convert_type(d2, jnp.int32)

    k = rn_ref[...]

    def step(t, r):
        cand = r + jnp.left_shift(jnp.int32(1), 30 - t)
        cnt = jnp.sum((xbits < cand).astype(jnp.int32), axis=1, keepdims=True)
        return jnp.where(cnt <= k, cand, r)

    r = jax.lax.fori_loop(0, 31, step, jnp.zeros((_BLK, 1), jnp.int32))
    chosen_ref[...] = jnp.sqrt(jax.lax.bitcast_convert_type(r, jnp.float32))

    dmask = diag.astype(jnp.float32)
    dd = jnp.sum(dot * dmask, axis=1, keepdims=True)
    n2d = jnp.sum(n2 * dmask, axis=1, keepdims=True)
    pos_ref[...] = n1 + n2d - 2.0 * dd


@functools.partial(jax.jit, static_argnames=())
def _run(output1, output2, rn):
    o2t = output2.T
    chosen, pos = pl.pallas_call(
        _body,
        grid=(_N // _BLK,),
        in_specs=[
            pl.BlockSpec((_BLK, _D), lambda i: (i, 0)),
            pl.BlockSpec((_D, _N), lambda i: (0, 0)),
            pl.BlockSpec((_BLK, 1), lambda i: (i, 0)),
        ],
        out_specs=[
            pl.BlockSpec((_BLK, 1), lambda i: (i, 0)),
            pl.BlockSpec((_BLK, 1), lambda i: (i, 0)),
        ],
        out_shape=[
            jax.ShapeDtypeStruct((_N, 1), jnp.float32),
            jax.ShapeDtypeStruct((_N, 1), jnp.float32),
        ],
    )(output1, o2t, rn)
    neg_loss = jnp.clip(2.0 - chosen[:, 0], 0.0, None)
    return jnp.mean(pos[:, 0]) + jnp.mean(neg_loss)


def kernel(output1, output2, quant):
    N = output1.shape[0]
    q = min(100, N - 1)
    rn = jax.random.randint(jax.random.key(1234), (N,), 0, q)
    rn = jnp.minimum(rn, quant - 1).astype(jnp.int32)[:, None]
    return _run(output1, output2, rn)
